# baseline (speedup 1.0000x reference)
import functools

import jax
import jax.numpy as jnp
from jax import lax
from jax.experimental import pallas as pl
from jax.experimental.pallas import tpu as pltpu
from jax.experimental.pallas import tpu_sc as plsc

N = 10000
E = 320000
D = 128

NC = 2
NS = 16
NW = NC * NS
K = 128
CH = -(-E // (NW * K))
E_PAD = NW * CH * K
RPS = -(-(-(-N // NS)) // 8) * 8
N_PAD = NS * RPS
_PIECES = []
_o = 0
while _o < RPS:
    _PIECES.append((_o, min(K, RPS - _o)))
    _o += K


def _sc_body(src_hbm, dst_hbm, w_hbm, x2_hbm, part_hbm,
             src_v, dst_v, w_v, rows_v, acc, sem):
    c = lax.axis_index("c")
    s = lax.axis_index("s")
    wid = s * NC + c

    pltpu.sync_copy(src_hbm.at[wid], src_v)
    pltpu.sync_copy(dst_hbm.at[wid], dst_v)
    pltpu.sync_copy(w_hbm.at[wid], w_v)

    def _zero(j, carry):
        for l in range(D // 16):
            rows_v[j, pl.ds(l * 16, 16)] = jnp.zeros((16,), jnp.float32)
        return carry

    lax.fori_loop(0, K, _zero, 0)
    base = s * RPS
    for off, sz in _PIECES:
        pltpu.sync_copy(rows_v.at[pl.ds(0, sz)], acc.at[pl.ds(base + off, sz)])
    plsc.subcore_barrier()

    def _chunk(ci, carry):
        pltpu.async_copy(x2_hbm.at[2 * c + s % 2].at[src_v.at[ci]], rows_v, sem).wait()

        def _scale(g, carry2):
            wvec = w_v[ci, pl.ds(g * 16, 16)]
            for j2 in range(16):
                j = g * 16 + j2
                ws = wvec[j2]
                for l in range(D // 16):
                    rows_v[j, pl.ds(l * 16, 16)] = (
                        rows_v[j, pl.ds(l * 16, 16)] * ws)
            return carry2

        lax.fori_loop(0, K // 16, _scale, 0)
        pltpu.sync_copy(rows_v, acc.at[dst_v.at[ci]], add=True)
        return carry

    lax.fori_loop(0, CH, _chunk, 0)
    plsc.subcore_barrier()

    for off, sz in _PIECES:
        pltpu.sync_copy(acc.at[pl.ds(base + off, sz)], rows_v.at[pl.ds(0, sz)])
        pltpu.sync_copy(rows_v.at[pl.ds(0, sz)],
                        part_hbm.at[c, pl.ds(base + off, sz)])


_sc_aggregate = functools.partial(
    pl.kernel,
    out_type=jax.ShapeDtypeStruct((NC, N_PAD, D), jnp.float32),
    mesh=plsc.VectorSubcoreMesh(
        core_axis_name="c", subcore_axis_name="s",
        num_cores=NC, num_subcores=NS),
    scratch_types=[
        pltpu.VMEM((CH, K), jnp.int32),
        pltpu.VMEM((CH, K), jnp.int32),
        pltpu.VMEM((CH, K), jnp.float32),
        pltpu.VMEM((K, D), jnp.float32),
        pltpu.VMEM_SHARED((N_PAD, D), jnp.float32),
        pltpu.SemaphoreType.DMA,
    ],
)(_sc_body)


def _tc_body(p0_ref, p1_ref, w_ref, o_ref):
    z = p0_ref[...] + p1_ref[...]
    o_ref[...] = jnp.maximum(
        jnp.dot(z, w_ref[...], preferred_element_type=jnp.float32), 0.0)


_TC_BLK = 2000


def _tc_combine(p0, p1, W):
    return pl.pallas_call(
        _tc_body,
        grid=(N // _TC_BLK,),
        in_specs=[
            pl.BlockSpec((_TC_BLK, D), lambda i: (i, 0)),
            pl.BlockSpec((_TC_BLK, D), lambda i: (i, 0)),
            pl.BlockSpec((D, D), lambda i: (0, 0)),
        ],
        out_specs=pl.BlockSpec((_TC_BLK, D), lambda i: (i, 0)),
        out_shape=jax.ShapeDtypeStruct((N, D), jnp.float32),
    )(p0, p1, W)


@jax.jit
def kernel(x, edge_index, edge_weight, W):
    pad = E_PAD - E
    src = jnp.concatenate(
        [edge_index[1], jnp.zeros((pad,), jnp.int32)]).reshape(NW, CH, K)
    dst = jnp.concatenate(
        [edge_index[0], jnp.zeros((pad,), jnp.int32)]).reshape(NW, CH, K)
    w = jnp.concatenate(
        [edge_weight, jnp.zeros((pad,), jnp.float32)]).reshape(NW, CH, K)
    x2 = jnp.stack([x, x, x, x])
    part = _sc_aggregate(src, dst, w, x2)
    return _tc_combine(part[0, :N], part[1, :N], W)


# speedup vs baseline: 1.1607x; 1.1607x over previous
import functools

import jax
import jax.numpy as jnp
from jax import lax
from jax.experimental import pallas as pl
from jax.experimental.pallas import tpu as pltpu
from jax.experimental.pallas import tpu_sc as plsc

N = 10000
E = 320000
D = 128

NC = 2
NS = 16
NW = NC * NS
K = 128
CH = -(-E // (NW * K))
E_PAD = NW * CH * K
RPS = -(-(-(-N // NS)) // 8) * 8
N_PAD = NS * RPS
_PIECES = []
_o = 0
while _o < RPS:
    _PIECES.append((_o, min(K, RPS - _o)))
    _o += K


def _sc_body(src_hbm, dst_hbm, w_hbm, x2_hbm, part_hbm,
             src_v, dst_v, w_v, rows_v, acc, sem):
    c = lax.axis_index("c")
    s = lax.axis_index("s")
    wid = s * NC + c

    pltpu.sync_copy(src_hbm.at[wid], src_v)
    pltpu.sync_copy(dst_hbm.at[wid], dst_v)
    pltpu.sync_copy(w_hbm.at[wid], w_v)

    def _zero(j, carry):
        for l in range(D // 16):
            rows_v[j, pl.ds(l * 16, 16)] = jnp.zeros((16,), jnp.float32)
        return carry

    lax.fori_loop(0, K, _zero, 0)
    base = s * RPS
    for off, sz in _PIECES:
        pltpu.sync_copy(rows_v.at[pl.ds(0, sz)], acc.at[pl.ds(base + off, sz)])
    plsc.subcore_barrier()

    def _chunk(ci, carry):
        pltpu.async_copy(x2_hbm.at[c].at[src_v.at[ci]], rows_v, sem).wait()

        def _scale(g, carry2):
            wvec = w_v[ci, pl.ds(g * 16, 16)]
            for j2 in range(16):
                j = g * 16 + j2
                ws = wvec[j2]
                for l in range(D // 16):
                    rows_v[j, pl.ds(l * 16, 16)] = (
                        rows_v[j, pl.ds(l * 16, 16)] * ws)
            return carry2

        lax.fori_loop(0, K // 16, _scale, 0)
        pltpu.sync_copy(rows_v, acc.at[dst_v.at[ci]], add=True)
        return carry

    lax.fori_loop(0, CH, _chunk, 0)
    plsc.subcore_barrier()

    for off, sz in _PIECES:
        pltpu.sync_copy(acc.at[pl.ds(base + off, sz)], rows_v.at[pl.ds(0, sz)])
        pltpu.sync_copy(rows_v.at[pl.ds(0, sz)],
                        part_hbm.at[c, pl.ds(base + off, sz)])


_sc_aggregate = functools.partial(
    pl.kernel,
    out_type=jax.ShapeDtypeStruct((NC, N_PAD, D), jnp.float32),
    mesh=plsc.VectorSubcoreMesh(
        core_axis_name="c", subcore_axis_name="s",
        num_cores=NC, num_subcores=NS),
    scratch_types=[
        pltpu.VMEM((CH, K), jnp.int32),
        pltpu.VMEM((CH, K), jnp.int32),
        pltpu.VMEM((CH, K), jnp.float32),
        pltpu.VMEM((K, D), jnp.float32),
        pltpu.VMEM_SHARED((N_PAD, D), jnp.float32),
        pltpu.SemaphoreType.DMA,
    ],
)(_sc_body)


def _tc_body(p0_ref, p1_ref, w_ref, o_ref):
    z = p0_ref[...] + p1_ref[...]
    o_ref[...] = jnp.maximum(
        jnp.dot(z, w_ref[...], preferred_element_type=jnp.float32), 0.0)


_TC_BLK = 2000


def _tc_combine(p0, p1, W):
    return pl.pallas_call(
        _tc_body,
        grid=(N // _TC_BLK,),
        in_specs=[
            pl.BlockSpec((_TC_BLK, D), lambda i: (i, 0)),
            pl.BlockSpec((_TC_BLK, D), lambda i: (i, 0)),
            pl.BlockSpec((D, D), lambda i: (0, 0)),
        ],
        out_specs=pl.BlockSpec((_TC_BLK, D), lambda i: (i, 0)),
        out_shape=jax.ShapeDtypeStruct((N, D), jnp.float32),
    )(p0, p1, W)


@jax.jit
def kernel(x, edge_index, edge_weight, W):
    pad = E_PAD - E
    src = jnp.concatenate(
        [edge_index[1], jnp.zeros((pad,), jnp.int32)]).reshape(NW, CH, K)
    dst = jnp.concatenate(
        [edge_index[0], jnp.zeros((pad,), jnp.int32)]).reshape(NW, CH, K)
    w = jnp.concatenate(
        [edge_weight, jnp.zeros((pad,), jnp.float32)]).reshape(NW, CH, K)
    x2 = jnp.stack([x, x])
    part = _sc_aggregate(src, dst, w, x2)
    return _tc_combine(part[0, :N], part[1, :N], W)
